# Initial kernel scaffold; baseline (speedup 1.0000x reference)
import functools

import jax
import jax.numpy as jnp
import numpy as np
from jax import lax
from jax.experimental import pallas as pl
from jax.experimental.pallas import tpu as pltpu
from jax.experimental.pallas import tpu_sc as plsc

_N = 256
_NOS = 512
_M = 4
_SIGMA = 2.0
_B = 2
_F = 4
_P = 8192
_PTS = _B * _P
_BWIN = (2.0 - 1.0 / _SIGMA) * np.pi


def _build_mats():
    a = np.arange(_N, dtype=np.float64)
    s = np.arange(_NOS, dtype=np.float64)
    t = np.i0(_M * np.sqrt(_BWIN ** 2 - (2.0 * np.pi * (a - _N / 2) / _NOS) ** 2))
    ph = -2.0 * np.pi * np.outer(s + _NOS / 2, a - _N / 2) / _NOS
    Ar = (np.cos(ph) / t[None, :]).astype(np.float32)
    Ai = (np.sin(ph) / t[None, :]).astype(np.float32)
    B2a = np.zeros((_N, 2 * _NOS), np.float32)
    B2b = np.zeros((_N, 2 * _NOS), np.float32)
    B2a[:, 0::2] = Ar.T
    B2a[:, 1::2] = Ai.T
    B2b[:, 0::2] = -Ai.T
    B2b[:, 1::2] = Ar.T
    return Ar, Ai, B2a, B2b


_AR, _AI, _B2A, _B2B = _build_mats()


def _dense_body(fh_ref, ar_ref, ai_ref, b2a_ref, b2b_ref, out_ref):
    fhb = fh_ref[0]
    t1r = jnp.dot(ar_ref[...], fhb, preferred_element_type=jnp.float32)
    t1i = jnp.dot(ai_ref[...], fhb, preferred_element_type=jnp.float32)
    out_ref[0] = (jnp.dot(t1r, b2a_ref[...], preferred_element_type=jnp.float32)
                  + jnp.dot(t1i, b2b_ref[...], preferred_element_type=jnp.float32))


def _dense_stage(f_hat):
    fh = f_hat.reshape(_B * _F, _N, _N)
    grid = (_B * _F,)
    return pl.pallas_call(
        _dense_body,
        grid=grid,
        in_specs=[
            pl.BlockSpec((1, _N, _N), lambda c: (c, 0, 0)),
            pl.BlockSpec((_NOS, _N), lambda c: (0, 0)),
            pl.BlockSpec((_NOS, _N), lambda c: (0, 0)),
            pl.BlockSpec((_N, 2 * _NOS), lambda c: (0, 0)),
            pl.BlockSpec((_N, 2 * _NOS), lambda c: (0, 0)),
        ],
        out_specs=pl.BlockSpec((1, _NOS, 2 * _NOS), lambda c: (c, 0, 0)),
        out_shape=jax.ShapeDtypeStruct((_B * _F, _NOS, 2 * _NOS), jnp.float32),
    )(fh, jnp.asarray(_AR), jnp.asarray(_AI), jnp.asarray(_B2A), jnp.asarray(_B2B))


def _weights_body(xt_ref, wt_ref, idx_ref):
    taps = lax.broadcasted_iota(jnp.int32, (8, 1), 0)
    for b in range(_B):
        x0 = xt_ref[2 * b, :]
        x1 = xt_ref[2 * b + 1, :]

        def win(xc):
            base = jnp.ceil(xc * _NOS).astype(jnp.int32) - _M
            k = xc[None, :] - (base[None, :] + taps).astype(jnp.float32) / _NOS
            u = (_NOS * k) ** 2
            arg = jnp.sqrt(jnp.maximum(_M * _M - u, 0.0))
            ba = _BWIN * arg
            sh = 0.5 * (jnp.exp(ba) - jnp.exp(-ba))
            val = sh / (arg * np.pi)
            w = jnp.where(u < float(_M * _M), val, 0.0)
            pos = jnp.mod(base[None, :] + taps + 768, _NOS)
            return w, pos

        wx, ri = win(x0)
        wy, ci = win(x1)
        wt = (wx[:, None, :] * wy[None, :, :]).reshape(64, _P)
        idx = (b * _NOS * _NOS + ri[:, None, :] * _NOS
               + ci[None, :, :]).reshape(64, _P)
        wt_ref[:, b * _P:(b + 1) * _P] = wt
        idx_ref[:, b * _P:(b + 1) * _P] = idx


def _weights_stage(x):
    xt = x.reshape(_B, _P, 2).transpose(0, 2, 1).reshape(2 * _B, _P)
    return pl.pallas_call(
        _weights_body,
        in_specs=[pl.BlockSpec((2 * _B, _P), lambda: (0, 0))],
        out_specs=[
            pl.BlockSpec((64, _PTS), lambda: (0, 0)),
            pl.BlockSpec((64, _PTS), lambda: (0, 0)),
        ],
        out_shape=[
            jax.ShapeDtypeStruct((64, _PTS), jnp.float32),
            jax.ShapeDtypeStruct((64, _PTS), jnp.int32),
        ],
    )(xt)


_NTILES = 32
_PPT = _PTS // _NTILES
_CH = 128
_NCHUNK = _PPT // _CH


def _sc_body(table_hbm, idx_hbm, wt_hbm, out_hbm, idx_v, wt_v, rows_v, out_v, sem):
    nc = 2
    wid = lax.axis_index("s") * nc + lax.axis_index("c")
    base_pt = wid * _PPT
    lane = lax.iota(jnp.int32, 16)

    for chunk in range(_NCHUNK):
        coff = base_pt + chunk * _CH
        pltpu.sync_copy(idx_hbm.at[:, pl.ds(coff, _CH)], idx_v)
        pltpu.sync_copy(wt_hbm.at[:, pl.ds(coff, _CH)], wt_v)
        copies = []
        for w in range(64):
            copies.append(
                pltpu.async_copy(table_hbm.at[idx_v.at[w]], rows_v.at[w], sem))
        for c in copies:
            c.wait()
        for g in range(_CH // 16):
            pvec = lane + (g * 16)

            def wbody(w, accs):
                wtv = wt_v[w, pl.ds(g * 16, 16)]
                wsp = jnp.full((16,), w, jnp.int32)
                new = []
                for v in range(8):
                    vsp = jnp.full((16,), v, jnp.int32)
                    d = plsc.load_gather(rows_v, [wsp, pvec, vsp])
                    new.append(accs[v] + wtv * d)
                return tuple(new)

            accs = lax.fori_loop(
                0, 64, wbody, tuple(jnp.zeros((16,), jnp.float32) for _ in range(8)))
            for v in range(8):
                out_v[v, pl.ds(chunk * _CH + g * 16, 16)] = accs[v]

    pltpu.sync_copy(out_v, out_hbm.at[:, pl.ds(base_pt, _PPT)])


def _sc_stage(table, idx, wt):
    mesh = plsc.VectorSubcoreMesh(core_axis_name="c", subcore_axis_name="s")
    kern = functools.partial(
        pl.kernel,
        mesh=mesh,
        out_type=jax.ShapeDtypeStruct((8, _PTS), jnp.float32),
        scratch_types=[
            pltpu.VMEM((64, _CH), jnp.int32),
            pltpu.VMEM((64, _CH), jnp.float32),
            pltpu.VMEM((64, _CH, 8), jnp.float32),
            pltpu.VMEM((8, _PPT), jnp.float32),
            pltpu.SemaphoreType.DMA,
        ],
    )(_sc_body)
    return kern(table, idx, wt)


def kernel(x, f_hat):
    gpair = _dense_stage(f_hat)
    table = (gpair.reshape(_B, _F, _NOS, _NOS, 2)
             .transpose(0, 2, 3, 1, 4)
             .reshape(_B * _NOS * _NOS, 2 * _F))
    wt, idx = _weights_stage(x)
    out = _sc_stage(table, idx, wt)
    o = out.reshape(_F, 2, _B, _P)
    return (o[:, 0] + 1j * o[:, 1]).astype(jnp.complex64).transpose(1, 0, 2)


# speedup vs baseline: 306.7621x; 306.7621x over previous
import functools

import jax
import jax.numpy as jnp
import numpy as np
from jax import lax
from jax.experimental import pallas as pl
from jax.experimental.pallas import tpu as pltpu
from jax.experimental.pallas import tpu_sc as plsc

_N = 256
_NOS = 512
_M = 4
_SIGMA = 2.0
_B = 2
_F = 4
_P = 8192
_PTS = _B * _P
_BWIN = (2.0 - 1.0 / _SIGMA) * np.pi


def _build_mats():
    a = np.arange(_N, dtype=np.float64)
    s = np.arange(_NOS, dtype=np.float64)
    t = np.i0(_M * np.sqrt(_BWIN ** 2 - (2.0 * np.pi * (a - _N / 2) / _NOS) ** 2))
    ph = -2.0 * np.pi * np.outer(s + _NOS / 2, a - _N / 2) / _NOS
    Ar = (np.cos(ph) / t[None, :]).astype(np.float32)
    Ai = (np.sin(ph) / t[None, :]).astype(np.float32)
    B2a = np.zeros((_N, 2 * _NOS), np.float32)
    B2b = np.zeros((_N, 2 * _NOS), np.float32)
    B2a[:, 0::2] = Ar.T
    B2a[:, 1::2] = Ai.T
    B2b[:, 0::2] = -Ai.T
    B2b[:, 1::2] = Ar.T
    return Ar, Ai, B2a, B2b


_AR, _AI, _B2A, _B2B = _build_mats()


def _dense_body(fh_ref, ar_ref, ai_ref, b2a_ref, b2b_ref, out_ref):
    fhb = fh_ref[0]
    t1r = jnp.dot(ar_ref[...], fhb, preferred_element_type=jnp.float32)
    t1i = jnp.dot(ai_ref[...], fhb, preferred_element_type=jnp.float32)
    out_ref[0] = (jnp.dot(t1r, b2a_ref[...], preferred_element_type=jnp.float32)
                  + jnp.dot(t1i, b2b_ref[...], preferred_element_type=jnp.float32))


def _dense_stage(f_hat):
    fh = f_hat.reshape(_B * _F, _N, _N)
    grid = (_B * _F,)
    return pl.pallas_call(
        _dense_body,
        grid=grid,
        in_specs=[
            pl.BlockSpec((1, _N, _N), lambda c: (c, 0, 0)),
            pl.BlockSpec((_NOS, _N), lambda c: (0, 0)),
            pl.BlockSpec((_NOS, _N), lambda c: (0, 0)),
            pl.BlockSpec((_N, 2 * _NOS), lambda c: (0, 0)),
            pl.BlockSpec((_N, 2 * _NOS), lambda c: (0, 0)),
        ],
        out_specs=pl.BlockSpec((1, _NOS, 2 * _NOS), lambda c: (c, 0, 0)),
        out_shape=jax.ShapeDtypeStruct((_B * _F, _NOS, 2 * _NOS), jnp.float32),
    )(fh, jnp.asarray(_AR), jnp.asarray(_AI), jnp.asarray(_B2A), jnp.asarray(_B2B))


def _weights_body(xt_ref, wt_ref, idx_ref):
    taps = lax.broadcasted_iota(jnp.int32, (8, 1), 0)
    for b in range(_B):
        x0 = xt_ref[2 * b, :]
        x1 = xt_ref[2 * b + 1, :]

        def win(xc):
            base = jnp.ceil(xc * _NOS).astype(jnp.int32) - _M
            k = xc[None, :] - (base[None, :] + taps).astype(jnp.float32) / _NOS
            u = (_NOS * k) ** 2
            arg = jnp.sqrt(jnp.maximum(_M * _M - u, 0.0))
            ba = _BWIN * arg
            sh = 0.5 * (jnp.exp(ba) - jnp.exp(-ba))
            val = sh / (arg * np.pi)
            w = jnp.where(u < float(_M * _M), val, 0.0)
            pos = jnp.mod(base[None, :] + taps + 768, _NOS)
            return w, pos

        wx, ri = win(x0)
        wy, ci = win(x1)
        wt = (wx[:, None, :] * wy[None, :, :]).reshape(64, _P)
        idx = (b * _NOS * _NOS + ri[:, None, :] * _NOS
               + ci[None, :, :]).reshape(64, _P)
        wt_ref[:, b * _P:(b + 1) * _P] = wt
        idx_ref[:, b * _P:(b + 1) * _P] = idx


def _weights_stage(x):
    xt = x.reshape(_B, _P, 2).transpose(0, 2, 1).reshape(2 * _B, _P)
    return pl.pallas_call(
        _weights_body,
        in_specs=[pl.BlockSpec((2 * _B, _P), lambda: (0, 0))],
        out_specs=[
            pl.BlockSpec((64, _PTS), lambda: (0, 0)),
            pl.BlockSpec((64, _PTS), lambda: (0, 0)),
        ],
        out_shape=[
            jax.ShapeDtypeStruct((64, _PTS), jnp.float32),
            jax.ShapeDtypeStruct((64, _PTS), jnp.int32),
        ],
    )(xt)


_NTILES = 32
_PPT = _PTS // _NTILES
_CH = 128
_NCHUNK = _PPT // _CH


def _sc_body(table_hbm, idx_hbm, wt_hbm, out_hbm, idx_v, wt_v, rows_v, out_v, sem):
    nc = 2
    wid = lax.axis_index("s") * nc + lax.axis_index("c")
    base_pt = wid * _PPT
    lane = lax.iota(jnp.int32, 16)

    for chunk in range(_NCHUNK):
        coff = base_pt + chunk * _CH
        pltpu.sync_copy(idx_hbm.at[:, pl.ds(coff, _CH)], idx_v)
        pltpu.sync_copy(wt_hbm.at[:, pl.ds(coff, _CH)], wt_v)
        copies = []
        for w in range(64):
            copies.append(
                pltpu.async_copy(table_hbm.at[idx_v.at[w]],
                                 rows_v.at[pl.ds(w * _CH, _CH)], sem))
        for c in copies:
            c.wait()
        for g in range(_CH // 16):
            pvec = lane + (g * 16)

            def wbody(w, accs):
                wtv = wt_v[w, pl.ds(g * 16, 16)]
                rvec = pvec + w * _CH
                new = []
                for v in range(8):
                    vsp = jnp.full((16,), v, jnp.int32)
                    d = plsc.load_gather(rows_v, [rvec, vsp])
                    new.append(accs[v] + wtv * d)
                return tuple(new)

            accs = lax.fori_loop(
                0, 64, wbody, tuple(jnp.zeros((16,), jnp.float32) for _ in range(8)))
            for v in range(8):
                out_v[v, pl.ds(chunk * _CH + g * 16, 16)] = accs[v]

    pltpu.sync_copy(out_v, out_hbm.at[:, pl.ds(base_pt, _PPT)])


def _sc_stage(table, idx, wt):
    mesh = plsc.VectorSubcoreMesh(core_axis_name="c", subcore_axis_name="s")
    kern = functools.partial(
        pl.kernel,
        mesh=mesh,
        out_type=jax.ShapeDtypeStruct((8, _PTS), jnp.float32),
        compiler_params=pltpu.CompilerParams(needs_layout_passes=False,
                                             use_tc_tiling_on_sc=False),
        scratch_types=[
            pltpu.VMEM((64, _CH), jnp.int32),
            pltpu.VMEM((64, _CH), jnp.float32),
            pltpu.VMEM((64 * _CH, 8), jnp.float32),
            pltpu.VMEM((8, _PPT), jnp.float32),
            pltpu.SemaphoreType.DMA,
        ],
    )(_sc_body)
    return kern(table, idx, wt)


def kernel(x, f_hat):
    gpair = _dense_stage(f_hat)
    table = (gpair.reshape(_B, _F, _NOS, _NOS, 2)
             .transpose(0, 2, 3, 1, 4)
             .reshape(_B * _NOS * _NOS, 2 * _F))
    wt, idx = _weights_stage(x)
    out = _sc_stage(table, idx, wt)
    o = out.reshape(_F, 2, _B, _P)
    return (o[:, 0] + 1j * o[:, 1]).astype(jnp.complex64).transpose(1, 0, 2)


# speedup vs baseline: 638.6749x; 2.0820x over previous
import functools

import jax
import jax.numpy as jnp
import numpy as np
from jax import lax
from jax.experimental import pallas as pl
from jax.experimental.pallas import tpu as pltpu
from jax.experimental.pallas import tpu_sc as plsc

_N = 256
_NOS = 512
_M = 4
_SIGMA = 2.0
_B = 2
_F = 4
_P = 8192
_PTS = _B * _P
_BWIN = (2.0 - 1.0 / _SIGMA) * np.pi


def _build_mats():
    a = np.arange(_N, dtype=np.float64)
    s = np.arange(_NOS, dtype=np.float64)
    t = np.i0(_M * np.sqrt(_BWIN ** 2 - (2.0 * np.pi * (a - _N / 2) / _NOS) ** 2))
    ph = -2.0 * np.pi * np.outer(s + _NOS / 2, a - _N / 2) / _NOS
    Ar = (np.cos(ph) / t[None, :]).astype(np.float32)
    Ai = (np.sin(ph) / t[None, :]).astype(np.float32)
    Bbig = np.zeros((2 * _F * _N, 8 * _NOS), np.float32)
    cols = 8 * np.arange(_NOS)
    for f in range(_F):
        rr = slice((2 * f + 0) * _N, (2 * f + 1) * _N)
        ri = slice((2 * f + 1) * _N, (2 * f + 2) * _N)
        Bbig[rr, cols + 2 * f] = Ar.T
        Bbig[rr, cols + 2 * f + 1] = Ai.T
        Bbig[ri, cols + 2 * f] = -Ai.T
        Bbig[ri, cols + 2 * f + 1] = Ar.T
    return Ar, Ai, Bbig


_AR, _AI, _BBIG = _build_mats()


def _t1_body(fh_ref, ar_ref, ai_ref, out_ref):
    parts = []
    for f in range(_F):
        fhb = fh_ref[0, f]
        parts.append(jnp.dot(ar_ref[...], fhb, preferred_element_type=jnp.float32))
        parts.append(jnp.dot(ai_ref[...], fhb, preferred_element_type=jnp.float32))
    out_ref[0] = jnp.concatenate(parts, axis=1)


def _t1_stage(f_hat):
    return pl.pallas_call(
        _t1_body,
        grid=(_B,),
        in_specs=[
            pl.BlockSpec((1, _F, _N, _N), lambda b: (b, 0, 0, 0)),
            pl.BlockSpec((_NOS, _N), lambda b: (0, 0)),
            pl.BlockSpec((_NOS, _N), lambda b: (0, 0)),
        ],
        out_specs=pl.BlockSpec((1, _NOS, 2 * _F * _N), lambda b: (b, 0, 0)),
        out_shape=jax.ShapeDtypeStruct((_B, _NOS, 2 * _F * _N), jnp.float32),
    )(f_hat, jnp.asarray(_AR), jnp.asarray(_AI))


def _tab_body(t1_ref, bb_ref, out_ref):
    y = jnp.dot(t1_ref[0], bb_ref[...], preferred_element_type=jnp.float32)
    out_ref[0] = y.reshape(_NOS, 8, 128)


def _tab_stage(t1cat):
    nj = 8 * _NOS // 1024
    return pl.pallas_call(
        _tab_body,
        grid=(nj, _B),
        in_specs=[
            pl.BlockSpec((1, _NOS, 2 * _F * _N), lambda j, b: (b, 0, 0)),
            pl.BlockSpec((2 * _F * _N, 1024), lambda j, b: (0, j)),
        ],
        out_specs=pl.BlockSpec((1, _NOS, 8, 128), lambda j, b: (b, 0, j, 0)),
        out_shape=jax.ShapeDtypeStruct((_B, _NOS, 8 * nj, 128), jnp.float32),
    )(t1cat, jnp.asarray(_BBIG))


def _weights_body(xt_ref, wt_ref, idx_ref):
    taps = lax.broadcasted_iota(jnp.int32, (8, 1, 1), 0)
    for b in range(_B):
        x0 = xt_ref[2 * b]
        x1 = xt_ref[2 * b + 1]

        def win(xc):
            base = jnp.ceil(xc * _NOS).astype(jnp.int32) - _M
            k = xc[None] - (base[None] + taps).astype(jnp.float32) / _NOS
            u = (_NOS * k) ** 2
            arg = jnp.sqrt(jnp.maximum(_M * _M - u, 0.0))
            ba = _BWIN * arg
            sh = 0.5 * (jnp.exp(ba) - jnp.exp(-ba))
            val = sh / (arg * np.pi)
            w = jnp.where(u < float(_M * _M), val, 0.0)
            pos = jnp.mod(base[None] + taps + 768, _NOS)
            return w, pos

        wx, ri = win(x0)
        wy, ci = win(x1)
        wt = (wx[:, None] * wy[None]).reshape(64, 64, 128)
        idx = (b * _NOS * _NOS + ri[:, None] * _NOS
               + ci[None]).reshape(64, 64, 128)
        wt_ref[:, b * 64:(b + 1) * 64, :] = wt
        idx_ref[:, b * 64:(b + 1) * 64, :] = idx


def _weights_stage(x):
    xt = x.reshape(_B, _P, 2).transpose(0, 2, 1).reshape(2 * _B, _P // 128, 128)
    return pl.pallas_call(
        _weights_body,
        in_specs=[pl.BlockSpec((2 * _B, _P // 128, 128), lambda: (0, 0, 0))],
        out_specs=[
            pl.BlockSpec((64, _PTS // 128, 128), lambda: (0, 0, 0)),
            pl.BlockSpec((64, _PTS // 128, 128), lambda: (0, 0, 0)),
        ],
        out_shape=[
            jax.ShapeDtypeStruct((64, _PTS // 128, 128), jnp.float32),
            jax.ShapeDtypeStruct((64, _PTS // 128, 128), jnp.int32),
        ],
    )(xt)


_NTILES = 32
_PPT = _PTS // _NTILES
_CH = 128
_NCHUNK = _PPT // _CH


def _sc_body(table_hbm, idx_hbm, wt_hbm, out_hbm, idx_v, wt_v, rows_v, out_v, sem):
    nc = 2
    wid = lax.axis_index("s") * nc + lax.axis_index("c")
    base_pt = wid * _PPT
    lane = lax.iota(jnp.int32, 16)

    for chunk in range(_NCHUNK):
        coff = base_pt + chunk * _CH
        pltpu.sync_copy(idx_hbm.at[:, pl.ds(coff, _CH)], idx_v)
        pltpu.sync_copy(wt_hbm.at[:, pl.ds(coff, _CH)], wt_v)
        copies = []
        for w in range(64):
            copies.append(
                pltpu.async_copy(table_hbm.at[idx_v.at[w]],
                                 rows_v.at[pl.ds(w * _CH, _CH)], sem))
        for c in copies:
            c.wait()
        for g in range(_CH // 16):
            pvec = lane + (g * 16)

            def wbody(w, accs):
                wtv = wt_v[w, pl.ds(g * 16, 16)]
                rvec = pvec + w * _CH
                new = []
                for v in range(8):
                    vsp = jnp.full((16,), v, jnp.int32)
                    d = plsc.load_gather(rows_v, [rvec, vsp])
                    new.append(accs[v] + wtv * d)
                return tuple(new)

            accs = lax.fori_loop(
                0, 64, wbody, tuple(jnp.zeros((16,), jnp.float32) for _ in range(8)))
            for v in range(8):
                out_v[v, pl.ds(chunk * _CH + g * 16, 16)] = accs[v]

    pltpu.sync_copy(out_v, out_hbm.at[:, pl.ds(base_pt, _PPT)])


def _sc_stage(table, idx, wt):
    mesh = plsc.VectorSubcoreMesh(core_axis_name="c", subcore_axis_name="s")
    kern = functools.partial(
        pl.kernel,
        mesh=mesh,
        out_type=jax.ShapeDtypeStruct((8, _PTS), jnp.float32),
        compiler_params=pltpu.CompilerParams(needs_layout_passes=False,
                                             use_tc_tiling_on_sc=False),
        scratch_types=[
            pltpu.VMEM((64, _CH), jnp.int32),
            pltpu.VMEM((64, _CH), jnp.float32),
            pltpu.VMEM((64 * _CH, 8), jnp.float32),
            pltpu.VMEM((8, _PPT), jnp.float32),
            pltpu.SemaphoreType.DMA,
        ],
    )(_sc_body)
    return kern(table, idx, wt)


def kernel(x, f_hat):
    t1cat = _t1_stage(f_hat)
    tab4 = _tab_stage(t1cat)
    table = tab4.reshape(_B * _NOS * _NOS, 2 * _F)
    wt4, idx4 = _weights_stage(x)
    wt = wt4.reshape(64, _PTS)
    idx = idx4.reshape(64, _PTS)
    out = _sc_stage(table, idx, wt)
    o = out.reshape(_F, 2, _B, _P)
    return (o[:, 0] + 1j * o[:, 1]).astype(jnp.complex64).transpose(1, 0, 2)


# speedup vs baseline: 687.8577x; 1.0770x over previous
import functools

import jax
import jax.numpy as jnp
import numpy as np
from jax import lax
from jax.experimental import pallas as pl
from jax.experimental.pallas import tpu as pltpu
from jax.experimental.pallas import tpu_sc as plsc

_N = 256
_NOS = 512
_M = 4
_SIGMA = 2.0
_B = 2
_F = 4
_P = 8192
_PTS = _B * _P
_BWIN = (2.0 - 1.0 / _SIGMA) * np.pi


def _build_mats():
    a = np.arange(_N, dtype=np.float64)
    s = np.arange(_NOS, dtype=np.float64)
    t = np.i0(_M * np.sqrt(_BWIN ** 2 - (2.0 * np.pi * (a - _N / 2) / _NOS) ** 2))
    ph = -2.0 * np.pi * np.outer(s + _NOS / 2, a - _N / 2) / _NOS
    Ar = (np.cos(ph) / t[None, :]).astype(np.float32)
    Ai = (np.sin(ph) / t[None, :]).astype(np.float32)
    Bbig = np.zeros((2 * _F * _N, 8 * _NOS), np.float32)
    cols = 8 * np.arange(_NOS)
    for f in range(_F):
        rr = slice((2 * f + 0) * _N, (2 * f + 1) * _N)
        ri = slice((2 * f + 1) * _N, (2 * f + 2) * _N)
        Bbig[rr, cols + 2 * f] = Ar.T
        Bbig[rr, cols + 2 * f + 1] = Ai.T
        Bbig[ri, cols + 2 * f] = -Ai.T
        Bbig[ri, cols + 2 * f + 1] = Ar.T
    return Ar, Ai, Bbig


_AR, _AI, _BBIG = _build_mats()


def _t1_body(fh_ref, ar_ref, ai_ref, out_ref):
    parts = []
    for f in range(_F):
        fhb = fh_ref[0, f]
        parts.append(jnp.dot(ar_ref[...], fhb, preferred_element_type=jnp.float32))
        parts.append(jnp.dot(ai_ref[...], fhb, preferred_element_type=jnp.float32))
    out_ref[0] = jnp.concatenate(parts, axis=1)


def _t1_stage(f_hat):
    return pl.pallas_call(
        _t1_body,
        grid=(_B,),
        in_specs=[
            pl.BlockSpec((1, _F, _N, _N), lambda b: (b, 0, 0, 0)),
            pl.BlockSpec((_NOS, _N), lambda b: (0, 0)),
            pl.BlockSpec((_NOS, _N), lambda b: (0, 0)),
        ],
        out_specs=pl.BlockSpec((1, _NOS, 2 * _F * _N), lambda b: (b, 0, 0)),
        out_shape=jax.ShapeDtypeStruct((_B, _NOS, 2 * _F * _N), jnp.float32),
    )(f_hat, jnp.asarray(_AR), jnp.asarray(_AI))


def _tab_body(t1_ref, bb_ref, out_ref):
    y = jnp.dot(t1_ref[0], bb_ref[...], preferred_element_type=jnp.float32)
    out_ref[0] = y.reshape(_NOS, 8, 128)


def _tab_stage(t1cat):
    nj = 8 * _NOS // 1024
    return pl.pallas_call(
        _tab_body,
        grid=(nj, _B),
        in_specs=[
            pl.BlockSpec((1, _NOS, 2 * _F * _N), lambda j, b: (b, 0, 0)),
            pl.BlockSpec((2 * _F * _N, 1024), lambda j, b: (0, j)),
        ],
        out_specs=pl.BlockSpec((1, _NOS, 8, 128), lambda j, b: (b, 0, j, 0)),
        out_shape=jax.ShapeDtypeStruct((_B, _NOS, 8 * nj, 128), jnp.float32),
    )(t1cat, jnp.asarray(_BBIG))


def _weights_body(xt_ref, wt_ref, idx_ref):
    taps = lax.broadcasted_iota(jnp.int32, (8, 1, 1), 0)
    for b in range(_B):
        x0 = xt_ref[2 * b]
        x1 = xt_ref[2 * b + 1]

        def win(xc):
            base = jnp.ceil(xc * _NOS).astype(jnp.int32) - _M
            k = xc[None] - (base[None] + taps).astype(jnp.float32) / _NOS
            u = (_NOS * k) ** 2
            arg = jnp.sqrt(jnp.maximum(_M * _M - u, 0.0))
            ba = _BWIN * arg
            sh = 0.5 * (jnp.exp(ba) - jnp.exp(-ba))
            val = sh / (arg * np.pi)
            w = jnp.where(u < float(_M * _M), val, 0.0)
            pos = jnp.mod(base[None] + taps + 768, _NOS)
            return w, pos

        wx, ri = win(x0)
        wy, ci = win(x1)
        wt = (wx[:, None] * wy[None]).reshape(64, 64, 128)
        idx = (b * _NOS * _NOS + ri[:, None] * _NOS
               + ci[None]).reshape(64, 64, 128)
        wt_ref[:, b * 64:(b + 1) * 64, :] = wt
        idx_ref[:, b * 64:(b + 1) * 64, :] = idx


def _weights_stage(x):
    xt = x.reshape(_B, _P, 2).transpose(0, 2, 1).reshape(2 * _B, _P // 128, 128)
    return pl.pallas_call(
        _weights_body,
        in_specs=[pl.BlockSpec((2 * _B, _P // 128, 128), lambda: (0, 0, 0))],
        out_specs=[
            pl.BlockSpec((64, _PTS // 128, 128), lambda: (0, 0, 0)),
            pl.BlockSpec((64, _PTS // 128, 128), lambda: (0, 0, 0)),
        ],
        out_shape=[
            jax.ShapeDtypeStruct((64, _PTS // 128, 128), jnp.float32),
            jax.ShapeDtypeStruct((64, _PTS // 128, 128), jnp.int32),
        ],
    )(xt)


_NTILES = 32
_PPT = _PTS // _NTILES


def _vperm(a, idx):
    dnums = lax.GatherDimensionNumbers(
        offset_dims=(), collapsed_slice_dims=(0,), start_index_map=(0,))
    return lax.gather(a, idx[:, None], dnums, (1,),
                      mode=lax.GatherScatterMode.PROMISE_IN_BOUNDS)
_CH = 64
_NCHUNK = _PPT // _CH


def _sc_body(table_hbm, idx_hbm, wt_hbm, out_hbm,
             idx_a, idx_b, rows_a, rows_b, wt_full, out_v, sem_a, sem_b):
    nc = 2
    wid = lax.axis_index("s") * nc + lax.axis_index("c")
    base_pt = wid * _PPT
    lane = lax.iota(jnp.int32, 16)
    upper = lane >> 3
    low3 = lane & 7
    rks = [upper + 2 * k for k in range(8)]

    pltpu.sync_copy(wt_hbm.at[:, pl.ds(base_pt, _PPT)], wt_full)

    def fire(c):
        idx_v, rows_v, sem = (idx_a, rows_a, sem_a) if c % 2 == 0 else (
            idx_b, rows_b, sem_b)
        pltpu.sync_copy(idx_hbm.at[:, pl.ds(base_pt + c * _CH, _CH)], idx_v)
        return [pltpu.async_copy(table_hbm.at[idx_v.at[w]],
                                 rows_v.at[pl.ds(w * _CH, _CH)], sem)
                for w in range(64)]

    def compute(c):
        rows_v = rows_a if c % 2 == 0 else rows_b

        def gbody(g, _):
            goff = g * 16

            def wbody(w, accs):
                wtv = wt_full[w, pl.ds(c * _CH + goff, 16)]
                s = w * _CH + goff
                new = []
                for k in range(8):
                    d = plsc.load_gather(rows_v, [rks[k] + s, low3])
                    pw = _vperm(wtv, rks[k])
                    new.append(accs[k] + pw * d)
                return tuple(new)

            accs = plsc.parallel_loop(
                0, 64, unroll=2,
                carry=tuple(jnp.zeros((16,), jnp.float32) for _ in range(8)),
            )(wbody)
            for k in range(8):
                plsc.store_scatter(out_v, [low3, upper + (c * _CH + goff + 2 * k)],
                                   accs[k])
            return 0

        lax.fori_loop(0, _CH // 16, gbody, 0)

    pend = fire(0)
    for c in range(_NCHUNK):
        nxt = fire(c + 1) if c + 1 < _NCHUNK else []
        for cp in pend:
            cp.wait()
        pend = nxt
        compute(c)

    pltpu.sync_copy(out_v, out_hbm.at[:, pl.ds(base_pt, _PPT)])


def _sc_stage(table, idx, wt):
    mesh = plsc.VectorSubcoreMesh(core_axis_name="c", subcore_axis_name="s")
    kern = functools.partial(
        pl.kernel,
        mesh=mesh,
        out_type=jax.ShapeDtypeStruct((8, _PTS), jnp.float32),
        compiler_params=pltpu.CompilerParams(needs_layout_passes=False,
                                             use_tc_tiling_on_sc=False),
        scratch_types=[
            pltpu.VMEM((64, _CH), jnp.int32),
            pltpu.VMEM((64, _CH), jnp.int32),
            pltpu.VMEM((64 * _CH, 8), jnp.float32),
            pltpu.VMEM((64 * _CH, 8), jnp.float32),
            pltpu.VMEM((64, _PPT), jnp.float32),
            pltpu.VMEM((8, _PPT), jnp.float32),
            pltpu.SemaphoreType.DMA,
            pltpu.SemaphoreType.DMA,
        ],
    )(_sc_body)
    return kern(table, idx, wt)


def kernel(x, f_hat):
    t1cat = _t1_stage(f_hat)
    tab4 = _tab_stage(t1cat)
    table = tab4.reshape(_B * _NOS * _NOS, 2 * _F)
    wt4, idx4 = _weights_stage(x)
    wt = wt4.reshape(64, _PTS)
    idx = idx4.reshape(64, _PTS)
    out = _sc_stage(table, idx, wt)
    o = out.reshape(_F, 2, _B, _P)
    return (o[:, 0] + 1j * o[:, 1]).astype(jnp.complex64).transpose(1, 0, 2)


# speedup vs baseline: 779.5023x; 1.1332x over previous
import functools

import jax
import jax.numpy as jnp
import numpy as np
from jax import lax
from jax.experimental import pallas as pl
from jax.experimental.pallas import tpu as pltpu
from jax.experimental.pallas import tpu_sc as plsc

_N = 256
_NOS = 512
_M = 4
_SIGMA = 2.0
_B = 2
_F = 4
_P = 8192
_PTS = _B * _P
_BWIN = (2.0 - 1.0 / _SIGMA) * np.pi


def _build_mats():
    a = np.arange(_N, dtype=np.float64)
    s = np.arange(_NOS, dtype=np.float64)
    t = np.i0(_M * np.sqrt(_BWIN ** 2 - (2.0 * np.pi * (a - _N / 2) / _NOS) ** 2))
    ph = -2.0 * np.pi * np.outer(s + _NOS / 2, a - _N / 2) / _NOS
    Ar = (np.cos(ph) / t[None, :]).astype(np.float32)
    Ai = (np.sin(ph) / t[None, :]).astype(np.float32)
    Bbig = np.zeros((2 * _F * _N, 8 * _NOS), np.float32)
    cols = 8 * np.arange(_NOS)
    for f in range(_F):
        rr = slice((2 * f + 0) * _N, (2 * f + 1) * _N)
        ri = slice((2 * f + 1) * _N, (2 * f + 2) * _N)
        Bbig[rr, cols + 2 * f] = Ar.T
        Bbig[rr, cols + 2 * f + 1] = Ai.T
        Bbig[ri, cols + 2 * f] = -Ai.T
        Bbig[ri, cols + 2 * f + 1] = Ar.T
    return Ar, Ai, Bbig


_AR, _AI, _BBIG = _build_mats()


def _t1_body(fh_ref, ar_ref, ai_ref, out_ref):
    parts = []
    for f in range(_F):
        fhb = fh_ref[0, f]
        parts.append(jnp.dot(ar_ref[...], fhb, preferred_element_type=jnp.float32))
        parts.append(jnp.dot(ai_ref[...], fhb, preferred_element_type=jnp.float32))
    out_ref[0] = jnp.concatenate(parts, axis=1)


def _t1_stage(f_hat):
    return pl.pallas_call(
        _t1_body,
        grid=(_B,),
        in_specs=[
            pl.BlockSpec((1, _F, _N, _N), lambda b: (b, 0, 0, 0)),
            pl.BlockSpec((_NOS, _N), lambda b: (0, 0)),
            pl.BlockSpec((_NOS, _N), lambda b: (0, 0)),
        ],
        out_specs=pl.BlockSpec((1, _NOS, 2 * _F * _N), lambda b: (b, 0, 0)),
        out_shape=jax.ShapeDtypeStruct((_B, _NOS, 2 * _F * _N), jnp.float32),
    )(f_hat, jnp.asarray(_AR), jnp.asarray(_AI))


def _tab_body(t1_ref, bb_ref, out_ref):
    t1b = t1_ref[0].astype(jnp.bfloat16)
    y = jnp.dot(t1b, bb_ref[...], preferred_element_type=jnp.float32)
    out_ref[0] = y.reshape(_NOS, 8, 128)


def _tab_stage(t1cat):
    nj = 8 * _NOS // 1024
    return pl.pallas_call(
        _tab_body,
        grid=(nj, _B),
        in_specs=[
            pl.BlockSpec((1, _NOS, 2 * _F * _N), lambda j, b: (b, 0, 0)),
            pl.BlockSpec((2 * _F * _N, 1024), lambda j, b: (0, j)),
        ],
        out_specs=pl.BlockSpec((1, _NOS, 8, 128), lambda j, b: (b, 0, j, 0)),
        out_shape=jax.ShapeDtypeStruct((_B, _NOS, 8 * nj, 128), jnp.float32),
    )(t1cat, jnp.asarray(_BBIG, dtype=jnp.bfloat16))


def _weights_body(xt_ref, wt_ref, idx_ref):
    taps = lax.broadcasted_iota(jnp.int32, (8, 1, 1), 0)
    for b in range(_B):
        x0 = xt_ref[2 * b]
        x1 = xt_ref[2 * b + 1]

        def win(xc):
            base = jnp.ceil(xc * _NOS).astype(jnp.int32) - _M
            k = xc[None] - (base[None] + taps).astype(jnp.float32) / _NOS
            u = (_NOS * k) ** 2
            arg = jnp.sqrt(jnp.maximum(_M * _M - u, 0.0))
            ba = _BWIN * arg
            sh = 0.5 * (jnp.exp(ba) - jnp.exp(-ba))
            val = sh / (arg * np.pi)
            w = jnp.where(u < float(_M * _M), val, 0.0)
            pos = jnp.mod(base[None] + taps + 768, _NOS)
            return w, pos

        wx, ri = win(x0)
        wy, ci = win(x1)
        wt = (wx[:, None] * wy[None]).reshape(64, 64, 128)
        idx = (b * _NOS * _NOS + ri[:, None] * _NOS
               + ci[None]).reshape(64, 64, 128)
        wt_ref[:, b * 64:(b + 1) * 64, :] = wt
        idx_ref[:, b * 64:(b + 1) * 64, :] = idx


def _weights_stage(x):
    xt = x.reshape(_B, _P, 2).transpose(0, 2, 1).reshape(2 * _B, _P // 128, 128)
    return pl.pallas_call(
        _weights_body,
        in_specs=[pl.BlockSpec((2 * _B, _P // 128, 128), lambda: (0, 0, 0))],
        out_specs=[
            pl.BlockSpec((64, _PTS // 128, 128), lambda: (0, 0, 0)),
            pl.BlockSpec((64, _PTS // 128, 128), lambda: (0, 0, 0)),
        ],
        out_shape=[
            jax.ShapeDtypeStruct((64, _PTS // 128, 128), jnp.float32),
            jax.ShapeDtypeStruct((64, _PTS // 128, 128), jnp.int32),
        ],
    )(xt)


_NTILES = 32
_PPT = _PTS // _NTILES


def _vperm(a, idx):
    dnums = lax.GatherDimensionNumbers(
        offset_dims=(), collapsed_slice_dims=(0,), start_index_map=(0,))
    return lax.gather(a, idx[:, None], dnums, (1,),
                      mode=lax.GatherScatterMode.PROMISE_IN_BOUNDS)
_CH = 64
_NCHUNK = _PPT // _CH


def _sc_body(table_hbm, idx_hbm, wt_hbm, out_hbm,
             idx_a, idx_b, rows_a, rows_b, wt_full, out_v, sem_a, sem_b):
    nc = 2
    wid = lax.axis_index("s") * nc + lax.axis_index("c")
    base_pt = wid * _PPT
    lane = lax.iota(jnp.int32, 16)
    upper = lane >> 3
    low3 = lane & 7
    rks = [upper + 2 * k for k in range(8)]

    pltpu.sync_copy(wt_hbm.at[:, pl.ds(base_pt, _PPT)], wt_full)

    def fire(c):
        idx_v, rows_v, sem = (idx_a, rows_a, sem_a) if c % 2 == 0 else (
            idx_b, rows_b, sem_b)
        pltpu.sync_copy(idx_hbm.at[:, pl.ds(base_pt + c * _CH, _CH)], idx_v)
        return [pltpu.async_copy(table_hbm.at[idx_v.at[w]],
                                 rows_v.at[pl.ds(w * _CH, _CH)], sem)
                for w in range(64)]

    def compute(c):
        rows_v = rows_a if c % 2 == 0 else rows_b

        def gbody(g, _):
            goff = g * 16

            def wbody(w, accs):
                wtv = wt_full[w, pl.ds(c * _CH + goff, 16)]
                s = w * _CH + goff
                new = []
                for k in range(8):
                    d = plsc.load_gather(rows_v, [rks[k] + s, low3])
                    pw = _vperm(wtv, rks[k])
                    new.append(accs[k] + pw * d)
                return tuple(new)

            accs = plsc.parallel_loop(
                0, 64, unroll=2,
                carry=tuple(jnp.zeros((16,), jnp.float32) for _ in range(8)),
            )(wbody)
            for k in range(8):
                plsc.store_scatter(out_v, [low3, upper + (c * _CH + goff + 2 * k)],
                                   accs[k])
            return 0

        lax.fori_loop(0, _CH // 16, gbody, 0)

    pend = fire(0)
    for c in range(_NCHUNK):
        nxt = fire(c + 1) if c + 1 < _NCHUNK else []
        for cp in pend:
            cp.wait()
        pend = nxt
        compute(c)

    pltpu.sync_copy(out_v, out_hbm.at[:, pl.ds(base_pt, _PPT)])


def _sc_stage(table, idx, wt):
    mesh = plsc.VectorSubcoreMesh(core_axis_name="c", subcore_axis_name="s")
    kern = functools.partial(
        pl.kernel,
        mesh=mesh,
        out_type=jax.ShapeDtypeStruct((8, _PTS), jnp.float32),
        compiler_params=pltpu.CompilerParams(needs_layout_passes=False,
                                             use_tc_tiling_on_sc=False),
        scratch_types=[
            pltpu.VMEM((64, _CH), jnp.int32),
            pltpu.VMEM((64, _CH), jnp.int32),
            pltpu.VMEM((64 * _CH, 8), jnp.float32),
            pltpu.VMEM((64 * _CH, 8), jnp.float32),
            pltpu.VMEM((64, _PPT), jnp.float32),
            pltpu.VMEM((8, _PPT), jnp.float32),
            pltpu.SemaphoreType.DMA,
            pltpu.SemaphoreType.DMA,
        ],
    )(_sc_body)
    return kern(table, idx, wt)


def kernel(x, f_hat):
    t1cat = _t1_stage(f_hat)
    tab4 = _tab_stage(t1cat)
    table = tab4.reshape(_B * _NOS * _NOS, 2 * _F)
    wt4, idx4 = _weights_stage(x)
    wt = wt4.reshape(64, _PTS)
    idx = idx4.reshape(64, _PTS)
    out = _sc_stage(table, idx, wt)
    o = out.reshape(_F, 2, _B, _P)
    return (o[:, 0] + 1j * o[:, 1]).astype(jnp.complex64).transpose(1, 0, 2)
